# Initial kernel scaffold; baseline (speedup 1.0000x reference)
import jax
import jax.numpy as jnp
from jax.experimental import pallas as pl


def kernel(feats, edge_index_r0, edge_index_r1, W_self0, b_self0, W_neigh0, W_self1, b_self1, W_neigh1, W_mlp0, b_mlp0, W_mlp1, b_mlp1):
    raise NotImplementedError("write your pallas kernel here")


# speedup vs baseline: 5.6535x; 5.6535x over previous
import functools

import jax
import jax.numpy as jnp
from jax import lax
from jax.experimental import pallas as pl
from jax.experimental.pallas import tpu as pltpu
from jax.experimental.pallas import tpu_sc as plsc

N = 10000
E = 320000
D = 128
NS = 16
NPAD = 10240
NODES_PER_TILE = NPAD // NS
CHUNK = 128
NROWS = E // CHUNK
DRAIN_ROWS = 160
BASE_CHUNKS = NROWS // NS
EXTRA = NROWS - BASE_CHUNKS * NS


def _zero_vmem_2d(ref, rows):
    def zrow(t, carry):
        i = t // (D // 16)
        j = (t % (D // 16)) * 16
        ref[i, pl.ds(j, 16)] = jnp.zeros((16,), jnp.float32)
        return carry
    lax.fori_loop(0, rows * (D // 16), zrow, None)


def _zero_vmem_1d(ref, n):
    def z(t, carry):
        ref[pl.ds(t * 16, 16)] = jnp.zeros((16,), jnp.float32)
        return carry
    lax.fori_loop(0, n // 16, z, None)


def _make_sc_agg(with_deg):
    mesh = plsc.VectorSubcoreMesh(core_axis_name="c", subcore_axis_name="s")
    out_type = [jax.ShapeDtypeStruct((2, NPAD, D), jnp.float32)]
    scratch = [
        pltpu.VMEM((CHUNK,), jnp.int32),
        pltpu.VMEM((CHUNK,), jnp.int32),
        pltpu.VMEM((CHUNK, D), jnp.float32),
        pltpu.VMEM((DRAIN_ROWS, D), jnp.float32),
        pltpu.VMEM_SHARED((NPAD, D), jnp.float32),
        pltpu.SemaphoreType.DMA,
    ]
    if with_deg:
        out_type.append(jax.ShapeDtypeStruct((2, NPAD), jnp.float32))
        scratch += [
            pltpu.VMEM((CHUNK,), jnp.float32),
            pltpu.VMEM((NODES_PER_TILE,), jnp.float32),
            pltpu.VMEM_SHARED((NPAD,), jnp.float32),
        ]

    def body(x_hbm, src_hbm, dst_hbm, *rest):
        if with_deg:
            (sums_hbm, degs_hbm, src_v, dst_v, rows_v, drain_v, acc_sh, sem,
             ones_v, deg_v, deg_sh) = rest
        else:
            sums_hbm, src_v, dst_v, rows_v, drain_v, acc_sh, sem = rest
        c = lax.axis_index("c")
        s = lax.axis_index("s")
        base = s * NODES_PER_TILE

        _zero_vmem_2d(drain_v, DRAIN_ROWS)
        for k in range(NODES_PER_TILE // DRAIN_ROWS):
            pltpu.sync_copy(drain_v,
                            acc_sh.at[pl.ds(base + k * DRAIN_ROWS, DRAIN_ROWS)])
        if with_deg:
            _zero_vmem_1d(deg_v, NODES_PER_TILE)
            pltpu.sync_copy(deg_v, deg_sh.at[pl.ds(base, NODES_PER_TILE)])
            def one(t, carry):
                ones_v[pl.ds(t * 16, 16)] = jnp.ones((16,), jnp.float32)
                return carry
            lax.fori_loop(0, CHUNK // 16, one, None)
        plsc.subcore_barrier()

        start = s * BASE_CHUNKS + jnp.minimum(s, EXTRA)
        nchunks = BASE_CHUNKS + jnp.where(s < EXTRA, 1, 0)

        def chunk_body(k, carry):
            j = start + k
            pltpu.sync_copy(src_hbm.at[c, j], src_v)
            pltpu.sync_copy(dst_hbm.at[c, j], dst_v)
            pltpu.async_copy(x_hbm.at[src_v], rows_v, sem).wait()
            pltpu.sync_copy(rows_v, acc_sh.at[dst_v], add=True)
            if with_deg:
                pltpu.sync_copy(ones_v, deg_sh.at[dst_v], add=True)
            return carry
        lax.fori_loop(0, nchunks, chunk_body, None)

        plsc.subcore_barrier()

        for k in range(NODES_PER_TILE // DRAIN_ROWS):
            r0 = base + k * DRAIN_ROWS
            pltpu.sync_copy(acc_sh.at[pl.ds(r0, DRAIN_ROWS)], drain_v)
            pltpu.sync_copy(drain_v, sums_hbm.at[c, pl.ds(r0, DRAIN_ROWS)])
        if with_deg:
            pltpu.sync_copy(deg_sh.at[pl.ds(base, NODES_PER_TILE)], deg_v)
            pltpu.sync_copy(deg_v, degs_hbm.at[c, pl.ds(base, NODES_PER_TILE)])

    return pl.kernel(body, out_type=tuple(out_type), mesh=mesh,
                     scratch_types=tuple(scratch))


_sc_agg_deg = _make_sc_agg(True)
_sc_agg = _make_sc_agg(False)

_BR = 400


def _dense_layer(x, s0, s1, d0, d1, W_self, b_self, W_neigh, W_mlp, b_mlp,
                 relu):
    dn = (((1,), (1,)), ((), ()))

    def body(x_r, s0_r, s1_r, d0_r, d1_r, Ws_r, bs_r, Wn_r, Wm_r, bm_r, o_r):
        neigh = (s0_r[...] / jnp.maximum(d0_r[...], 1.0)
                 + s1_r[...] / jnp.maximum(d1_r[...], 1.0))
        hs = lax.dot_general(x_r[...], Ws_r[...], dn,
                             preferred_element_type=jnp.float32)
        hn = lax.dot_general(neigh, Wn_r[...], dn,
                             preferred_element_type=jnp.float32)
        rel = 2.0 * hs + 2.0 * bs_r[...] + hn
        out = lax.dot_general(rel, Wm_r[...], dn,
                              preferred_element_type=jnp.float32) + bm_r[...]
        if relu:
            out = jnp.maximum(out, 0.0)
        o_r[...] = out

    row_spec = pl.BlockSpec((_BR, D), lambda i: (i, 0))
    deg_spec = pl.BlockSpec((_BR, 1), lambda i: (i, 0))
    w_spec = pl.BlockSpec((D, D), lambda i: (0, 0))
    b_spec = pl.BlockSpec((1, D), lambda i: (0, 0))
    return pl.pallas_call(
        body,
        grid=(N // _BR,),
        in_specs=[row_spec, row_spec, row_spec, deg_spec, deg_spec,
                  w_spec, b_spec, w_spec, w_spec, b_spec],
        out_specs=row_spec,
        out_shape=jax.ShapeDtypeStruct((N, D), jnp.float32),
    )(x, s0, s1, d0, d1, W_self, b_self.reshape(1, D), W_neigh, W_mlp,
      b_mlp.reshape(1, D))


def kernel(feats, edge_index_r0, edge_index_r1,
           W_self0, b_self0, W_neigh0,
           W_self1, b_self1, W_neigh1,
           W_mlp0, b_mlp0, W_mlp1, b_mlp1):
    src = jnp.stack([edge_index_r0[0], edge_index_r1[0]]
                    ).astype(jnp.int32).reshape(2, NROWS, CHUNK)
    dst = jnp.stack([edge_index_r0[1], edge_index_r1[1]]
                    ).astype(jnp.int32).reshape(2, NROWS, CHUNK)

    sums0, degs = _sc_agg_deg(feats, src, dst)
    d0 = degs[0, :N].reshape(N, 1)
    d1 = degs[1, :N].reshape(N, 1)
    h = _dense_layer(feats, sums0[0, :N], sums0[1, :N], d0, d1,
                     W_self0, b_self0, W_neigh0, W_mlp0, b_mlp0, relu=True)
    sums1, = _sc_agg(h, src, dst)
    out = _dense_layer(h, sums1[0, :N], sums1[1, :N], d0, d1,
                       W_self1, b_self1, W_neigh1, W_mlp1, b_mlp1, relu=False)
    return out
